# Initial kernel scaffold; baseline (speedup 1.0000x reference)
import jax
import jax.numpy as jnp
from jax.experimental import pallas as pl


def kernel(atoms, bonds, edges, W, b):
    raise NotImplementedError("write your pallas kernel here")


# speedup vs baseline: 3.2892x; 3.2892x over previous
import functools

import jax
import jax.numpy as jnp
from jax.experimental import pallas as pl
from jax.experimental.pallas import tpu as pltpu

_B, _A, _D = 1024, 60, 5
_FA, _FB, _CONV = 128, 16, 128
_M = 4


def _tc_body(edges_ref, atoms_ref, bonds_ref, wa_ref, wb_ref, bias_ref, out_ref):
    m = _M
    r = m * _A
    edges = edges_ref[...].reshape(r, _D)
    valid = edges >= 0
    mol_base = (jax.lax.broadcasted_iota(jnp.int32, (r, _D), 0) // _A) * _A
    gidx = jnp.where(valid, edges + mol_base, -1)

    col = jax.lax.broadcasted_iota(jnp.int32, (r, r), 1)
    row = jax.lax.broadcasted_iota(jnp.int32, (r, r), 0)
    c = (row == col).astype(jnp.float32)
    for d in range(_D):
        c = c + (gidx[:, d : d + 1] == col).astype(jnp.float32)

    atoms = atoms_ref[...].reshape(r, _FA)
    g = jnp.dot(c, atoms, preferred_element_type=jnp.float32)
    s_bond = jnp.sum(bonds_ref[...], axis=2).reshape(r, _FB)

    y = (
        jnp.dot(g, wa_ref[...], preferred_element_type=jnp.float32)
        + jnp.dot(s_bond, wb_ref[...], preferred_element_type=jnp.float32)
        + bias_ref[...]
    )
    y = jnp.maximum(y, 0.0)

    deg = jnp.sum(valid.astype(jnp.int32), axis=1, keepdims=True)
    out = jnp.zeros((r, _CONV), dtype=jnp.float32)
    for t in range(_D):
        sel = (deg == t + 1).astype(jnp.float32)
        out = out + sel * y[:, t * _CONV : (t + 1) * _CONV]
    out_ref[...] = out.reshape(m, _A, _CONV)


@jax.jit
def kernel(atoms, bonds, edges, W, b):
    w_all = W.transpose(1, 0, 2).reshape(_FA + _FB, _D * _CONV)
    w_atom = w_all[:_FA]
    w_bond = w_all[_FA:]
    bias = b.reshape(1, _D * _CONV)

    grid = (_B // _M,)
    return pl.pallas_call(
        _tc_body,
        grid=grid,
        in_specs=[
            pl.BlockSpec((_M, _A, _D), lambda i: (i, 0, 0)),
            pl.BlockSpec((_M, _A, _FA), lambda i: (i, 0, 0)),
            pl.BlockSpec((_M, _A, _D, _FB), lambda i: (i, 0, 0, 0)),
            pl.BlockSpec((_FA, _D * _CONV), lambda i: (0, 0)),
            pl.BlockSpec((_FB, _D * _CONV), lambda i: (0, 0)),
            pl.BlockSpec((1, _D * _CONV), lambda i: (0, 0)),
        ],
        out_specs=pl.BlockSpec((_M, _A, _CONV), lambda i: (i, 0, 0)),
        out_shape=jax.ShapeDtypeStruct((_B, _A, _CONV), jnp.float32),
        compiler_params=pltpu.CompilerParams(
            dimension_semantics=("arbitrary",),
        ),
    )(edges, atoms, bonds, w_atom, w_bond, bias)


# speedup vs baseline: 4.6191x; 1.4044x over previous
import functools

import jax
import jax.numpy as jnp
from jax.experimental import pallas as pl
from jax.experimental.pallas import tpu as pltpu

_B, _A, _D = 1024, 60, 5
_FA, _FB, _CONV = 128, 16, 128
_M = 4


def _tc_body(edges_ref, atoms_ref, bonds_ref, wa_ref, wb_ref, bias_ref, out_ref):
    m = _M
    r = m * _A
    edges = edges_ref[...].reshape(r, _D)
    valid = edges >= 0
    mol_base = (jax.lax.broadcasted_iota(jnp.int32, (r, _D), 0) // _A) * _A
    gidx = jnp.where(valid, edges + mol_base, -1)

    col = jax.lax.broadcasted_iota(jnp.int32, (r, r), 1)
    c = (gidx[:, 0:1] == col).astype(jnp.float32)
    for d in range(1, _D):
        c = c + (gidx[:, d : d + 1] == col).astype(jnp.float32)

    atoms = atoms_ref[...].reshape(r, _FA)
    g = jnp.dot(c, atoms, preferred_element_type=jnp.float32) + atoms
    bonds = bonds_ref[...].reshape(r, _D * _FB)

    y = (
        jnp.dot(g, wa_ref[...], preferred_element_type=jnp.float32)
        + jnp.dot(bonds, wb_ref[...], preferred_element_type=jnp.float32)
        + bias_ref[...]
    )
    y = jnp.maximum(y, 0.0)

    deg = jnp.sum(valid.astype(jnp.int32), axis=1, keepdims=True)
    out = jnp.zeros((r, _CONV), dtype=jnp.float32)
    for t in range(_D):
        sel = (deg == t + 1).astype(jnp.float32)
        out = out + sel * y[:, t * _CONV : (t + 1) * _CONV]
    out_ref[...] = out.reshape(m, _A, _CONV)


@jax.jit
def kernel(atoms, bonds, edges, W, b):
    w_all = W.transpose(1, 0, 2).reshape(_FA + _FB, _D * _CONV)
    w_atom = w_all[:_FA]
    w_bond = jnp.tile(w_all[_FA:], (_D, 1))
    bias = b.reshape(1, _D * _CONV)
    bonds_flat = bonds.reshape(_B, _A, _D * _FB)

    grid = (_B // _M,)
    return pl.pallas_call(
        _tc_body,
        grid=grid,
        in_specs=[
            pl.BlockSpec((_M, _A, _D), lambda i: (i, 0, 0)),
            pl.BlockSpec((_M, _A, _FA), lambda i: (i, 0, 0)),
            pl.BlockSpec((_M, _A, _D * _FB), lambda i: (i, 0, 0)),
            pl.BlockSpec((_FA, _D * _CONV), lambda i: (0, 0)),
            pl.BlockSpec((_D * _FB, _D * _CONV), lambda i: (0, 0)),
            pl.BlockSpec((1, _D * _CONV), lambda i: (0, 0)),
        ],
        out_specs=pl.BlockSpec((_M, _A, _CONV), lambda i: (i, 0, 0)),
        out_shape=jax.ShapeDtypeStruct((_B, _A, _CONV), jnp.float32),
        compiler_params=pltpu.CompilerParams(
            dimension_semantics=("arbitrary",),
        ),
    )(edges, atoms, bonds_flat, w_atom, w_bond, bias)


# speedup vs baseline: 5.3603x; 1.1605x over previous
import functools

import jax
import jax.numpy as jnp
from jax.experimental import pallas as pl
from jax.experimental.pallas import tpu as pltpu

_B, _A, _D = 1024, 60, 5
_FA, _FB, _CONV = 128, 16, 128
_M = 8


def _tc_body(edges_ref, atoms_ref, bonds_ref, wa_ref, wb_ref, bias_ref, out_ref):
    m = _M
    r = m * _A
    edges = edges_ref[...].reshape(r, _D)
    valid = edges >= 0
    mol_base = (jax.lax.broadcasted_iota(jnp.int32, (r, _D), 0) // _A) * _A
    gidx = jnp.where(valid, edges + mol_base, -1)

    col = jax.lax.broadcasted_iota(jnp.int32, (r, r), 1)
    c = (gidx[:, 0:1] == col).astype(jnp.float32)
    for d in range(1, _D):
        c = c + (gidx[:, d : d + 1] == col).astype(jnp.float32)

    atoms = atoms_ref[...].reshape(r, _FA)
    g = jnp.dot(c, atoms, preferred_element_type=jnp.float32) + atoms
    bonds = bonds_ref[...].reshape(r, _D * _FB)

    y = (
        jnp.dot(g, wa_ref[...], preferred_element_type=jnp.float32)
        + jnp.dot(bonds, wb_ref[...], preferred_element_type=jnp.float32)
        + bias_ref[...]
    )
    y = jnp.maximum(y, 0.0)

    deg = jnp.sum(valid.astype(jnp.int32), axis=1, keepdims=True)
    out = jnp.zeros((r, _CONV), dtype=jnp.float32)
    for t in range(_D):
        sel = (deg == t + 1).astype(jnp.float32)
        out = out + sel * y[:, t * _CONV : (t + 1) * _CONV]
    out_ref[...] = out.reshape(m, _A, _CONV)


@jax.jit
def kernel(atoms, bonds, edges, W, b):
    w_all = W.transpose(1, 0, 2).reshape(_FA + _FB, _D * _CONV)
    w_atom = w_all[:_FA]
    w_bond = jnp.tile(w_all[_FA:], (_D, 1))
    bias = b.reshape(1, _D * _CONV)
    bonds_flat = bonds.reshape(_B, _A, _D * _FB)

    grid = (_B // _M,)
    return pl.pallas_call(
        _tc_body,
        grid=grid,
        in_specs=[
            pl.BlockSpec((_M, _A, _D), lambda i: (i, 0, 0)),
            pl.BlockSpec((_M, _A, _FA), lambda i: (i, 0, 0)),
            pl.BlockSpec((_M, _A, _D * _FB), lambda i: (i, 0, 0)),
            pl.BlockSpec((_FA, _D * _CONV), lambda i: (0, 0)),
            pl.BlockSpec((_D * _FB, _D * _CONV), lambda i: (0, 0)),
            pl.BlockSpec((1, _D * _CONV), lambda i: (0, 0)),
        ],
        out_specs=pl.BlockSpec((_M, _A, _CONV), lambda i: (i, 0, 0)),
        out_shape=jax.ShapeDtypeStruct((_B, _A, _CONV), jnp.float32),
        compiler_params=pltpu.CompilerParams(
            dimension_semantics=("arbitrary",),
        ),
    )(edges, atoms, bonds_flat, w_atom, w_bond, bias)
